# baseline (speedup 1.0000x reference)
import jax
import jax.numpy as jnp
from jax.experimental import pallas as pl
from jax.experimental.pallas import tpu as pltpu

_B = 32
_C = 192
_H = 56
_W = 56
_N = _C * _H * _W
_K = int(round(_N * 0.1))
_BOOST_STRENGTH = 1.0
_NCHUNK = 12
_CP = _C // 2 // _NCHUNK


def _key_to_f32(m):
    return jax.lax.bitcast_convert_type(
        m ^ ((m >> 31) & jnp.int32(0x7FFFFFFF)), jnp.float32)


def _body(x_ref, bf_ref, out_ref, pk_ref):
    bf = bf_ref[...]
    kf = jnp.float32(_K)
    nf = jnp.float32(_N)
    x = x_ref[0]
    b = x * bf
    pk_ref[...] = jnp.concatenate([b[:_C // 2], b[_C // 2:]], axis=2)

    def count_ge(fmid):
        parts = []
        for g in range(_NCHUNK):
            blk = pk_ref[g * _CP:(g + 1) * _CP]
            m = jnp.where(blk >= fmid, jnp.float32(1.0), jnp.float32(0.0))
            parts.append(jnp.sum(m, axis=(0, 1)))
        while len(parts) > 1:
            nxt = [a + c for a, c in zip(parts[0::2], parts[1::2])]
            if len(parts) % 2:
                nxt.append(parts[-1])
            parts = nxt
        return jnp.sum(parts[0])

    def cond(carry):
        lo, hi = carry[0], carry[1]
        return lo < hi - jnp.int32(1)

    def step(carry):
        lo, hi, clo, chi = carry
        bis = (lo & hi) + ((lo ^ hi) >> 1)
        frac = (clo - kf) / jnp.maximum(clo - chi, jnp.float32(1.0))
        midf = jnp.float32(lo) + (jnp.float32(hi) - jnp.float32(lo)) * frac
        midf = jnp.clip(midf, jnp.float32(lo) + 1.0, jnp.float32(hi) - 1.0)
        sec = jnp.clip(midf.astype(jnp.int32), lo + jnp.int32(1),
                       hi - jnp.int32(1))
        pa = jnp.minimum(sec, bis)
        pb = jnp.maximum(sec, bis)
        ca = count_ge(_key_to_f32(pa))
        cb = count_ge(_key_to_f32(pb))
        nlo = jnp.where(cb >= kf, pb, jnp.where(ca >= kf, pa, lo))
        nclo = jnp.where(cb >= kf, cb, jnp.where(ca >= kf, ca, clo))
        nhi = jnp.where(cb < kf, jnp.where(ca < kf, pa, pb), hi)
        nchi = jnp.where(cb < kf, jnp.where(ca < kf, ca, cb), chi)
        nhi = jnp.where(ca == kf, pa + jnp.int32(1), nhi)
        nlo = jnp.where(ca == kf, pa, nlo)
        nhi = jnp.where(cb == kf, pb + jnp.int32(1), nhi)
        nlo = jnp.where(cb == kf, pb, nlo)
        return (nlo, nhi, nclo, nchi)

    lo_inf = jnp.int32(-2139095041)
    p0 = jnp.int32(0x3F8CCCCD)
    bmax = jnp.max(pk_ref[...])
    imax = jax.lax.bitcast_convert_type(bmax, jnp.int32)
    hi0 = (imax ^ ((imax >> 31) & jnp.int32(0x7FFFFFFF))) + jnp.int32(1)
    c0 = count_ge(jnp.float32(1.1))
    ok0 = c0 >= kf
    in_rng = p0 < hi0
    lo1 = jnp.where(ok0 & in_rng, p0, lo_inf)
    clo1 = jnp.where(ok0 & in_rng, c0, nf)
    hi1 = jnp.where((~ok0) & in_rng, p0, hi0)
    chi1 = jnp.where((~ok0) & in_rng, c0, jnp.float32(0.0))
    done0 = (c0 == kf) & in_rng
    hi1 = jnp.where(done0, p0 + jnp.int32(1), hi1)
    lo1 = jnp.where(done0, p0, lo1)
    thresh = jax.lax.while_loop(cond, step, (lo1, hi1, clo1, chi1))[0]
    ft = _key_to_f32(thresh)
    out_ref[0] = jnp.where(b >= ft, x, jnp.float32(0.0))


def kernel(x, dutyCycle):
    target_density = jnp.float32(float(_K) / float(_N))
    bf = jnp.exp((target_density - dutyCycle.reshape(_C)) * jnp.float32(_BOOST_STRENGTH))
    bf_full = jnp.broadcast_to(bf[:, None, None], (_C, _H, _W))
    return pl.pallas_call(
        _body,
        grid=(_B,),
        in_specs=[
            pl.BlockSpec((1, _C, _H, _W), lambda b: (b, 0, 0, 0)),
            pl.BlockSpec((_C, _H, _W), lambda b: (0, 0, 0)),
        ],
        out_specs=pl.BlockSpec((1, _C, _H, _W), lambda b: (b, 0, 0, 0)),
        out_shape=jax.ShapeDtypeStruct((_B, _C, _H, _W), jnp.float32),
        scratch_shapes=[pltpu.VMEM((_C // 2, _H, 2 * _W), jnp.float32)],
    )(x, bf_full)


# speedup vs baseline: 1.0986x; 1.0986x over previous
import jax
import jax.numpy as jnp
from jax.experimental import pallas as pl
from jax.experimental.pallas import tpu as pltpu

_B = 32
_C = 192
_H = 56
_W = 56
_N = _C * _H * _W
_K = int(round(_N * 0.1))
_BOOST_STRENGTH = 1.0
_NCHUNK = 12
_CP = _C // 2 // _NCHUNK


def _key_to_f32(m):
    return jax.lax.bitcast_convert_type(
        m ^ ((m >> 31) & jnp.int32(0x7FFFFFFF)), jnp.float32)


def _body(x_ref, bf_ref, out_ref, pk_ref):
    bf = bf_ref[...]
    kf = jnp.float32(_K)
    nf = jnp.float32(_N)
    x = x_ref[0]
    b = x * bf
    pk_ref[...] = jnp.concatenate([b[:_C // 2], b[_C // 2:]], axis=2)

    def count_ge(fmid):
        parts = []
        for g in range(_NCHUNK):
            blk = pk_ref[g * _CP:(g + 1) * _CP]
            m = jnp.where(blk >= fmid, jnp.float32(1.0), jnp.float32(0.0))
            parts.append(jnp.sum(m, axis=(0, 1)))
        while len(parts) > 1:
            nxt = [a + c for a, c in zip(parts[0::2], parts[1::2])]
            if len(parts) % 2:
                nxt.append(parts[-1])
            parts = nxt
        return jnp.sum(parts[0])

    def cond(carry):
        lo, hi = carry[0], carry[1]
        return lo < hi - jnp.int32(1)

    def step(carry):
        lo, hi, clo, chi, it = carry
        bis = (lo & hi) + ((lo ^ hi) >> 1)
        frac = (clo - kf) / jnp.maximum(clo - chi, jnp.float32(1.0))
        midf = jnp.float32(lo) + (jnp.float32(hi) - jnp.float32(lo)) * frac
        midf = jnp.clip(midf, jnp.float32(lo) + 1.0, jnp.float32(hi) - 1.0)
        interp = jnp.clip(midf.astype(jnp.int32), lo + jnp.int32(1),
                          hi - jnp.int32(1))
        mid = jnp.where(it % 2 == 0, interp, bis)
        cnt = count_ge(_key_to_f32(mid))
        ok = cnt >= kf
        done = cnt == kf
        nlo = jnp.where(ok, mid, lo)
        nclo = jnp.where(ok, cnt, clo)
        nhi = jnp.where(done, mid + jnp.int32(1), jnp.where(ok, hi, mid))
        nchi = jnp.where(ok, chi, cnt)
        return (nlo, nhi, nclo, nchi, it + jnp.int32(1))

    lo_inf = jnp.int32(-2139095041)
    p0 = jnp.int32(0x3F8CCCCD)
    bmax = jnp.max(pk_ref[...])
    imax = jax.lax.bitcast_convert_type(bmax, jnp.int32)
    hi0 = (imax ^ ((imax >> 31) & jnp.int32(0x7FFFFFFF))) + jnp.int32(1)
    c0 = count_ge(jnp.float32(1.1))
    ok0 = c0 >= kf
    in_rng = p0 < hi0
    lo1 = jnp.where(ok0 & in_rng, p0, lo_inf)
    clo1 = jnp.where(ok0 & in_rng, c0, nf)
    hi1 = jnp.where((~ok0) & in_rng, p0, hi0)
    chi1 = jnp.where((~ok0) & in_rng, c0, jnp.float32(0.0))
    done0 = (c0 == kf) & in_rng
    hi1 = jnp.where(done0, p0 + jnp.int32(1), hi1)
    lo1 = jnp.where(done0, p0, lo1)
    thresh = jax.lax.while_loop(
        cond, step, (lo1, hi1, clo1, chi1, jnp.int32(0)))[0]
    ft = _key_to_f32(thresh)
    out_ref[0] = jnp.where(b >= ft, x, jnp.float32(0.0))


def kernel(x, dutyCycle):
    target_density = jnp.float32(float(_K) / float(_N))
    bf = jnp.exp((target_density - dutyCycle.reshape(_C)) * jnp.float32(_BOOST_STRENGTH))
    bf_full = jnp.broadcast_to(bf[:, None, None], (_C, _H, _W))
    return pl.pallas_call(
        _body,
        grid=(_B,),
        in_specs=[
            pl.BlockSpec((1, _C, _H, _W), lambda b: (b, 0, 0, 0)),
            pl.BlockSpec((_C, _H, _W), lambda b: (0, 0, 0)),
        ],
        out_specs=pl.BlockSpec((1, _C, _H, _W), lambda b: (b, 0, 0, 0)),
        out_shape=jax.ShapeDtypeStruct((_B, _C, _H, _W), jnp.float32),
        scratch_shapes=[pltpu.VMEM((_C // 2, _H, 2 * _W), jnp.float32)],
    )(x, bf_full)
